# Initial kernel scaffold; baseline (speedup 1.0000x reference)
import jax
import jax.numpy as jnp
from jax.experimental import pallas as pl


def kernel(y, x, f_y, W1, b1, W2, b2, neighbors_index, neighbors_row_splits):
    raise NotImplementedError("write your pallas kernel here")


# speedup vs baseline: 3.7635x; 3.7635x over previous
import functools

import jax
import jax.numpy as jnp
from jax import lax
from jax.experimental import pallas as pl
from jax.experimental.pallas import tpu as pltpu
from jax.experimental.pallas import tpu_sc as plsc

NW = 32
LANES = 16
NCH = 392
ECH = 512
ECH2 = 512
TLE = 2048


def _cdiv(a, b):
    return (a + b - 1) // b


def _rup(a, b):
    return _cdiv(a, b) * b


def _wid():
    return lax.axis_index("s") * 2 + lax.axis_index("c")


def _sr(ref, i):
    return ref[pl.ds(i, LANES)][0]


def _mesh():
    return plsc.VectorSubcoreMesh(core_axis_name="c", subcore_axis_name="s")


_SC_PARAMS = pltpu.CompilerParams(needs_layout_passes=False,
                                 use_tc_tiling_on_sc=False)


def _gather_phase(y16, x16, nbrp, rsp, N, E, EP):
    EPW = EP // NW
    nchunks = EPW // ECH2
    RSP = _rup(N + 1, LANES)
    bits = []
    b = 1
    while b <= N:
        b *= 2
    while b >= 1:
        bits.append(b)
        b //= 2

    @functools.partial(
        pl.kernel,
        out_type=(jax.ShapeDtypeStruct((EP, 16), jnp.float32),
                  jax.ShapeDtypeStruct((EP, 16), jnp.float32)),
        mesh=_mesh(),
        compiler_params=_SC_PARAMS,
        scratch_types=[
            pltpu.VMEM((RSP,), jnp.int32),
            pltpu.VMEM((1, ECH2), jnp.int32),
            pltpu.VMEM((1, ECH2), jnp.int32),
            pltpu.VMEM((ECH2, 16), jnp.float32),
            pltpu.VMEM((ECH2, 16), jnp.float32),
            pltpu.SemaphoreType.DMA,
        ],
    )
    def k(y_hbm, x_hbm, nbr_hbm, rs_hbm, aggy_hbm, aggx_hbm,
          rsv, idxv, segv, rowsy, rowsx, sem):
        w = _wid()
        pltpu.sync_copy(rs_hbm.at[pl.ds(0, RSP)], rsv)
        iot = jnp.arange(LANES, dtype=jnp.int32)

        @pl.loop(0, nchunks)
        def _(t):
            base = (w * nchunks + t) * ECH2
            pltpu.sync_copy(nbr_hbm.at[pl.ds(base, ECH2)], idxv.at[0])
            cpy = pltpu.async_copy(y_hbm.at[idxv.at[0]], rowsy, sem)

            def gseg(g, _):
                e = jnp.minimum(base + g * LANES + iot, E - 1)
                pos = jnp.zeros((LANES,), jnp.int32)
                for bb in bits:
                    cand = pos + bb
                    rsc = plsc.load_gather(rsv, [jnp.minimum(cand, N)])
                    pos = jnp.where((cand <= N) & (rsc <= e), cand, pos)
                segv[0, pl.ds(g * LANES, LANES)] = pos
                return 0

            lax.fori_loop(0, ECH2 // LANES, gseg, 0)
            cpy.wait()
            pltpu.sync_copy(rowsy, aggy_hbm.at[pl.ds(base, ECH2)])
            pltpu.async_copy(x_hbm.at[segv.at[0]], rowsx, sem).wait()
            pltpu.sync_copy(rowsx, aggx_hbm.at[pl.ds(base, ECH2)])

    return k(y16, x16, nbrp, rsp)


def _tc_mlp(aggy, aggx, w1a, w1b, b1, w2, b2, EP):
    grid = EP // TLE

    def body(ya_ref, xa_ref, w1a_ref, w1b_ref, b1_ref, w2_ref, b2_ref, o_ref):
        hp = lax.dot_general(
            ya_ref[...], w1a_ref[...], (((1,), (0,)), ((), ())),
            precision=lax.Precision.HIGHEST,
            preferred_element_type=jnp.float32)
        hp = hp + lax.dot_general(
            xa_ref[...], w1b_ref[...], (((1,), (0,)), ((), ())),
            precision=lax.Precision.HIGHEST,
            preferred_element_type=jnp.float32)
        h = jax.nn.gelu(hp + b1_ref[...])
        o_ref[...] = lax.dot_general(
            h, w2_ref[...], (((1,), (0,)), ((), ())),
            precision=lax.Precision.HIGHEST,
            preferred_element_type=jnp.float32) + b2_ref[...]

    return pl.pallas_call(
        body,
        grid=(grid,),
        in_specs=[
            pl.BlockSpec((TLE, 16), lambda i: (i, 0)),
            pl.BlockSpec((TLE, 16), lambda i: (i, 0)),
            pl.BlockSpec((16, 64), lambda i: (0, 0)),
            pl.BlockSpec((16, 64), lambda i: (0, 0)),
            pl.BlockSpec((1, 64), lambda i: (0, 0)),
            pl.BlockSpec((64, 64), lambda i: (0, 0)),
            pl.BlockSpec((1, 64), lambda i: (0, 0)),
        ],
        out_specs=pl.BlockSpec((TLE, 64), lambda i: (i, 0)),
        out_shape=jax.ShapeDtypeStruct((EP, 64), jnp.float32),
    )(aggy, aggx, w1a, w1b, b1, w2, b2)


def _reduce(k1d, fy, nbrp, rsp, N, EP, NPW, NWPAD):
    NCC = NPW // NCH
    RSW = NCH + 48

    @functools.partial(
        pl.kernel,
        out_type=jax.ShapeDtypeStruct((NWPAD * 64,), jnp.float32),
        mesh=_mesh(),
        compiler_params=_SC_PARAMS,
        scratch_types=[
            pltpu.VMEM((RSW,), jnp.int32),
            pltpu.VMEM((ECH * 64,), jnp.float32),
            pltpu.VMEM((1, ECH + 16), jnp.int32),
            pltpu.VMEM((ECH + 16, 64), jnp.float32),
            pltpu.VMEM((NCH * 64,), jnp.float32),
            pltpu.SemaphoreType.DMA,
        ],
    )
    def k(k_hbm, fy_hbm, nbr_hbm, rs_hbm, out_hbm,
          rsv, kbuf, idxv, fybuf, obuf, sem):
        w = _wid()
        n0 = w * NPW
        z16 = jnp.zeros((LANES,), jnp.float32)

        @pl.loop(0, NCC)
        def _(ncc):
            cs = n0 + ncc * NCH
            bs = (cs // 16) * 16
            h0 = cs - bs
            pltpu.sync_copy(rs_hbm.at[pl.ds(bs, RSW)], rsv)
            e0 = _sr(rsv, h0)
            e1c = _sr(rsv, h0 + NCH)
            nchunks = lax.max((e1c - e0 + ECH - 1) // ECH, 1)

            def schunk(s, carry):
                base = e0 + s * ECH
                cend = jnp.minimum(base + ECH, e1c)
                ib = (base // 16) * 16
                pltpu.sync_copy(nbr_hbm.at[pl.ds(ib, ECH + 16)], idxv.at[0])
                pltpu.async_copy(fy_hbm.at[idxv.at[0]], fybuf, sem).wait()
                pltpu.sync_copy(k_hbm.at[pl.ds(base * 64, ECH * 64)], kbuf)

                def edge_acc(elo, ehi, acc):
                    def eb(e, ac):
                        ko = (e - base) * 64
                        fr = e - ib
                        a0, a1, a2, a3 = ac
                        a0 = a0 + kbuf[pl.ds(ko, 16)] * fybuf[fr, pl.ds(0, 16)]
                        a1 = a1 + kbuf[pl.ds(ko + 16, 16)] * fybuf[fr, pl.ds(16, 16)]
                        a2 = a2 + kbuf[pl.ds(ko + 32, 16)] * fybuf[fr, pl.ds(32, 16)]
                        a3 = a3 + kbuf[pl.ds(ko + 48, 16)] * fybuf[fr, pl.ds(48, 16)]
                        return (a0, a1, a2, a3)

                    return lax.fori_loop(elo, ehi, eb, acc)

                def wcond(c):
                    return (c[0] < cs + NCH) & (_sr(rsv, c[0] + 1 - bs) <= cend)

                def wbody(c):
                    nn = c[0]
                    a = _sr(rsv, nn - bs)
                    b = _sr(rsv, nn + 1 - bs)
                    acc = edge_acc(jnp.maximum(a, base), b, c[1:])
                    sc = (jnp.ones((LANES,), jnp.float32)
                          / jnp.full((LANES,),
                                     jnp.maximum(b - a, 1).astype(jnp.float32)))
                    o = (nn - cs) * 64
                    obuf[pl.ds(o, 16)] = acc[0] * sc
                    obuf[pl.ds(o + 16, 16)] = acc[1] * sc
                    obuf[pl.ds(o + 32, 16)] = acc[2] * sc
                    obuf[pl.ds(o + 48, 16)] = acc[3] * sc
                    return (nn + 1, z16, z16, z16, z16)

                c = lax.while_loop(wcond, wbody, (carry[0],) + carry[1:])
                nn = c[0]
                a = _sr(rsv, nn - bs)
                st = jnp.maximum(a, base)
                en = jnp.maximum(cend, st)
                acc = edge_acc(st, en, c[1:])
                return (nn,) + acc

            lax.fori_loop(0, nchunks, schunk, (cs, z16, z16, z16, z16))
            pltpu.sync_copy(obuf, out_hbm.at[pl.ds(cs * 64, NCH * 64)])

    return k(k1d, fy, nbrp, rsp)


def kernel(y, x, f_y, W1, b1, W2, b2, neighbors_index, neighbors_row_splits):
    N = y.shape[0]
    E = neighbors_index.shape[0]

    EP = _rup(E + 1024, NW * ECH2)
    assert EP % TLE == 0

    NPW = _rup(_cdiv(N, NW), NCH)
    NWPAD = NW * NPW

    nbr = neighbors_index.astype(jnp.int32)
    rs = neighbors_row_splits.astype(jnp.int32)

    y16 = jnp.pad(y.astype(jnp.float32), ((0, 0), (0, 13)))
    x16 = jnp.pad(x.astype(jnp.float32), ((0, 0), (0, 13)))
    nbrp = jnp.pad(nbr, (0, EP - E))
    rsp = jnp.pad(rs, (0, NWPAD + 416 - (N + 1)), constant_values=E)

    w1a = jnp.pad(W1[:3].astype(jnp.float32), ((0, 13), (0, 0)))
    w1b = jnp.pad(W1[3:].astype(jnp.float32), ((0, 13), (0, 0)))
    b1r = b1.astype(jnp.float32).reshape(1, 64)
    b2r = b2.astype(jnp.float32).reshape(1, 64)

    aggy, aggx = _gather_phase(y16, x16, nbrp, rsp, N, E, EP)
    kk = _tc_mlp(aggy, aggx, w1a, w1b, b1r, W2.astype(jnp.float32), b2r, EP)
    out_flat = _reduce(kk.reshape(EP * 64), f_y.astype(jnp.float32),
                       nbrp, rsp, N, EP, NPW, NWPAD)
    return out_flat.reshape(NWPAD, 64)[:N]


# speedup vs baseline: 7.8950x; 2.0978x over previous
import functools

import jax
import jax.numpy as jnp
from jax import lax
from jax.experimental import pallas as pl
from jax.experimental.pallas import tpu as pltpu
from jax.experimental.pallas import tpu_sc as plsc

NW = 32
LANES = 16
NCH = 392
ECH = 512
ECH2 = 512
TLE = 2048


def _cdiv(a, b):
    return (a + b - 1) // b


def _rup(a, b):
    return _cdiv(a, b) * b


def _wid():
    return lax.axis_index("s") * 2 + lax.axis_index("c")


def _sr(ref, i):
    return ref[pl.ds(i, LANES)][0]


def _mesh():
    return plsc.VectorSubcoreMesh(core_axis_name="c", subcore_axis_name="s")


_SC_PARAMS = pltpu.CompilerParams(needs_layout_passes=False,
                                 use_tc_tiling_on_sc=False)


def _gather_phase(y16, x16, nbrp, rsp, N, E, EP):
    EPW = EP // NW
    nchunks = EPW // ECH2
    RSP = _rup(N + 1, LANES)
    bits = []
    b = 1
    while b <= N:
        b *= 2
    while b >= 1:
        bits.append(b)
        b //= 2

    @functools.partial(
        pl.kernel,
        out_type=(jax.ShapeDtypeStruct((EP, 16), jnp.float32),
                  jax.ShapeDtypeStruct((EP, 16), jnp.float32)),
        mesh=_mesh(),
        compiler_params=_SC_PARAMS,
        scratch_types=[
            pltpu.VMEM((RSP,), jnp.int32),
            pltpu.VMEM((1, ECH2), jnp.int32),
            pltpu.VMEM((1, ECH2), jnp.int32),
            pltpu.VMEM((ECH2, 16), jnp.float32),
            pltpu.VMEM((ECH2, 16), jnp.float32),
            pltpu.SemaphoreType.DMA,
        ],
    )
    def k(y_hbm, x_hbm, nbr_hbm, rs_hbm, aggy_hbm, aggx_hbm,
          rsv, idxv, segv, rowsy, rowsx, sem):
        w = _wid()
        pltpu.sync_copy(rs_hbm.at[pl.ds(0, RSP)], rsv)
        iot = jnp.arange(LANES, dtype=jnp.int32)

        @pl.loop(0, nchunks)
        def _(t):
            base = (w * nchunks + t) * ECH2
            pltpu.sync_copy(nbr_hbm.at[pl.ds(base, ECH2)], idxv.at[0])
            cpy = pltpu.async_copy(y_hbm.at[idxv.at[0]], rowsy, sem)

            def gseg(g, _):
                e = jnp.minimum(base + g * LANES + iot, E - 1)
                pos = jnp.zeros((LANES,), jnp.int32)
                for bb in bits:
                    cand = pos + bb
                    rsc = plsc.load_gather(rsv, [jnp.minimum(cand, N)])
                    pos = jnp.where((cand <= N) & (rsc <= e), cand, pos)
                segv[0, pl.ds(g * LANES, LANES)] = pos
                return 0

            lax.fori_loop(0, ECH2 // LANES, gseg, 0)
            cpy.wait()
            pltpu.sync_copy(rowsy, aggy_hbm.at[pl.ds(base, ECH2)])
            pltpu.async_copy(x_hbm.at[segv.at[0]], rowsx, sem).wait()
            pltpu.sync_copy(rowsx, aggx_hbm.at[pl.ds(base, ECH2)])

    return k(y16, x16, nbrp, rsp)


def _tc_mlp(a8y, a8x, w1aB, w1bB, b1B, w2B, b2B, EP):
    grid = EP // TLE
    R8 = TLE // 8

    def body(ya_ref, xa_ref, w1a_ref, w1b_ref, b1_ref, w2_ref, b2_ref, o_ref):
        hp = lax.dot_general(
            ya_ref[...], w1a_ref[...], (((1,), (0,)), ((), ())),
            precision=lax.Precision.HIGHEST,
            preferred_element_type=jnp.float32)
        hp = hp + lax.dot_general(
            xa_ref[...], w1b_ref[...], (((1,), (0,)), ((), ())),
            precision=lax.Precision.HIGHEST,
            preferred_element_type=jnp.float32)
        h8 = jax.nn.gelu(hp + b1_ref[...])
        h2 = h8.reshape(TLE // 2, 128)
        o_ref[...] = lax.dot_general(
            h2, w2_ref[...], (((1,), (0,)), ((), ())),
            precision=lax.Precision.HIGHEST,
            preferred_element_type=jnp.float32) + b2_ref[...]

    return pl.pallas_call(
        body,
        grid=(grid,),
        in_specs=[
            pl.BlockSpec((R8, 128), lambda i: (i, 0)),
            pl.BlockSpec((R8, 128), lambda i: (i, 0)),
            pl.BlockSpec((128, 512), lambda i: (0, 0)),
            pl.BlockSpec((128, 512), lambda i: (0, 0)),
            pl.BlockSpec((1, 512), lambda i: (0, 0)),
            pl.BlockSpec((128, 128), lambda i: (0, 0)),
            pl.BlockSpec((1, 128), lambda i: (0, 0)),
        ],
        out_specs=pl.BlockSpec((TLE // 2, 128), lambda i: (i, 0)),
        out_shape=jax.ShapeDtypeStruct((EP // 2, 128), jnp.float32),
    )(a8y, a8x, w1aB, w1bB, b1B, w2B, b2B)


def _reduce(k1d, fy, nbrp, rsp, N, EP, NPW, NWPAD):
    NCC = NPW // NCH
    RSW = NCH + 48

    @functools.partial(
        pl.kernel,
        out_type=jax.ShapeDtypeStruct((NWPAD * 64,), jnp.float32),
        mesh=_mesh(),
        compiler_params=_SC_PARAMS,
        scratch_types=[
            pltpu.VMEM((RSW,), jnp.int32),
            pltpu.VMEM((ECH * 64,), jnp.float32),
            pltpu.VMEM((1, ECH + 16), jnp.int32),
            pltpu.VMEM((ECH + 16, 64), jnp.float32),
            pltpu.VMEM((NCH * 64,), jnp.float32),
            pltpu.SemaphoreType.DMA,
        ],
    )
    def k(k_hbm, fy_hbm, nbr_hbm, rs_hbm, out_hbm,
          rsv, kbuf, idxv, fybuf, obuf, sem):
        w = _wid()
        n0 = w * NPW
        z16 = jnp.zeros((LANES,), jnp.float32)

        @pl.loop(0, NCC)
        def _(ncc):
            cs = n0 + ncc * NCH
            bs = (cs // 16) * 16
            h0 = cs - bs
            pltpu.sync_copy(rs_hbm.at[pl.ds(bs, RSW)], rsv)
            e0 = _sr(rsv, h0)
            e1c = _sr(rsv, h0 + NCH)
            nchunks = lax.max((e1c - e0 + ECH - 1) // ECH, 1)

            def schunk(s, carry):
                base = e0 + s * ECH
                cend = jnp.minimum(base + ECH, e1c)
                ib = (base // 16) * 16
                pltpu.sync_copy(nbr_hbm.at[pl.ds(ib, ECH + 16)], idxv.at[0])
                pltpu.async_copy(fy_hbm.at[idxv.at[0]], fybuf, sem).wait()
                pltpu.sync_copy(k_hbm.at[pl.ds(base * 64, ECH * 64)], kbuf)

                def edge_acc(elo, ehi, acc):
                    def eb(e, ac):
                        ko = (e - base) * 64
                        fr = e - ib
                        a0, a1, a2, a3 = ac
                        a0 = a0 + kbuf[pl.ds(ko, 16)] * fybuf[fr, pl.ds(0, 16)]
                        a1 = a1 + kbuf[pl.ds(ko + 16, 16)] * fybuf[fr, pl.ds(16, 16)]
                        a2 = a2 + kbuf[pl.ds(ko + 32, 16)] * fybuf[fr, pl.ds(32, 16)]
                        a3 = a3 + kbuf[pl.ds(ko + 48, 16)] * fybuf[fr, pl.ds(48, 16)]
                        return (a0, a1, a2, a3)

                    return lax.fori_loop(elo, ehi, eb, acc)

                def wcond(c):
                    return (c[0] < cs + NCH) & (_sr(rsv, c[0] + 1 - bs) <= cend)

                def wbody(c):
                    nn = c[0]
                    a = _sr(rsv, nn - bs)
                    b = _sr(rsv, nn + 1 - bs)
                    acc = edge_acc(jnp.maximum(a, base), b, c[1:])
                    sc = (jnp.ones((LANES,), jnp.float32)
                          / jnp.full((LANES,),
                                     jnp.maximum(b - a, 1).astype(jnp.float32)))
                    o = (nn - cs) * 64
                    obuf[pl.ds(o, 16)] = acc[0] * sc
                    obuf[pl.ds(o + 16, 16)] = acc[1] * sc
                    obuf[pl.ds(o + 32, 16)] = acc[2] * sc
                    obuf[pl.ds(o + 48, 16)] = acc[3] * sc
                    return (nn + 1, z16, z16, z16, z16)

                c = lax.while_loop(wcond, wbody, (carry[0],) + carry[1:])
                nn = c[0]
                a = _sr(rsv, nn - bs)
                st = jnp.maximum(a, base)
                en = jnp.maximum(cend, st)
                acc = edge_acc(st, en, c[1:])
                return (nn,) + acc

            lax.fori_loop(0, nchunks, schunk, (cs, z16, z16, z16, z16))
            pltpu.sync_copy(obuf, out_hbm.at[pl.ds(cs * 64, NCH * 64)])

    return k(k1d, fy, nbrp, rsp)


def kernel(y, x, f_y, W1, b1, W2, b2, neighbors_index, neighbors_row_splits):
    N = y.shape[0]
    E = neighbors_index.shape[0]

    EP = _rup(E + 1024, NW * ECH2)
    assert EP % TLE == 0

    NPW = _rup(_cdiv(N, NW), NCH)
    NWPAD = NW * NPW

    nbr = neighbors_index.astype(jnp.int32)
    rs = neighbors_row_splits.astype(jnp.int32)

    y16 = jnp.pad(y.astype(jnp.float32), ((0, 0), (0, 13)))
    x16 = jnp.pad(x.astype(jnp.float32), ((0, 0), (0, 13)))
    nbrp = jnp.pad(nbr, (0, EP - E))
    rsp = jnp.pad(rs, (0, NWPAD + 416 - (N + 1)), constant_values=E)

    w1a = jnp.pad(W1[:3].astype(jnp.float32), ((0, 13), (0, 0)))
    w1b = jnp.pad(W1[3:].astype(jnp.float32), ((0, 13), (0, 0)))
    eye8 = jnp.eye(8, dtype=jnp.float32)
    eye2 = jnp.eye(2, dtype=jnp.float32)
    w1aB = jnp.kron(eye8, w1a)
    w1bB = jnp.kron(eye8, w1b)
    b1B = jnp.tile(b1.astype(jnp.float32), 8).reshape(1, 512)
    w2B = jnp.kron(eye2, W2.astype(jnp.float32))
    b2B = jnp.tile(b2.astype(jnp.float32), 2).reshape(1, 128)

    aggy, aggx = _gather_phase(y16, x16, nbrp, rsp, N, E, EP)
    a8y = aggy.reshape(EP // 8, 128)
    a8x = aggx.reshape(EP // 8, 128)
    kk = _tc_mlp(a8y, a8x, w1aB, w1bB, b1B, w2B, b2B, EP)
    out_flat = _reduce(kk.reshape(EP * 64), f_y.astype(jnp.float32),
                       nbrp, rsp, N, EP, NPW, NWPAD)
    return out_flat.reshape(NWPAD, 64)[:N]


# speedup vs baseline: 9.7535x; 1.2354x over previous
import functools

import jax
import jax.numpy as jnp
from jax import lax
from jax.experimental import pallas as pl
from jax.experimental.pallas import tpu as pltpu
from jax.experimental.pallas import tpu_sc as plsc

NW = 32
LANES = 16
NCH = 392
ECH = 512
ECH2 = 512
TLE = 2048


def _cdiv(a, b):
    return (a + b - 1) // b


def _rup(a, b):
    return _cdiv(a, b) * b


def _wid():
    return lax.axis_index("s") * 2 + lax.axis_index("c")


def _sr(ref, i):
    return ref[pl.ds(i, LANES)][0]


def _mesh():
    return plsc.VectorSubcoreMesh(core_axis_name="c", subcore_axis_name="s")


_SC_PARAMS = pltpu.CompilerParams(needs_layout_passes=False,
                                 use_tc_tiling_on_sc=False)


def _gather_phase(y16, x16, nbrp, rsp, N, E, EP):
    EPW = EP // NW
    nchunks = EPW // ECH2
    RSP = _rup(N + 1, LANES)
    bits = []
    b = 1
    while b <= N:
        b *= 2
    while b >= 1:
        bits.append(b)
        b //= 2

    @functools.partial(
        pl.kernel,
        out_type=(jax.ShapeDtypeStruct((EP, 16), jnp.float32),
                  jax.ShapeDtypeStruct((EP, 16), jnp.float32)),
        mesh=_mesh(),
        compiler_params=_SC_PARAMS,
        scratch_types=[
            pltpu.VMEM((RSP,), jnp.int32),
            pltpu.VMEM((1, ECH2), jnp.int32),
            pltpu.VMEM((1, ECH2), jnp.int32),
            pltpu.VMEM((ECH2, 16), jnp.float32),
            pltpu.VMEM((ECH2, 16), jnp.float32),
            pltpu.SemaphoreType.DMA,
        ],
    )
    def k(y_hbm, x_hbm, nbr_hbm, rs_hbm, aggy_hbm, aggx_hbm,
          rsv, idxv, segv, rowsy, rowsx, sem):
        w = _wid()
        pltpu.sync_copy(rs_hbm.at[pl.ds(0, RSP)], rsv)
        iot = jnp.arange(LANES, dtype=jnp.int32)

        @pl.loop(0, nchunks)
        def _(t):
            base = (w * nchunks + t) * ECH2
            pltpu.sync_copy(nbr_hbm.at[pl.ds(base, ECH2)], idxv.at[0])
            cpy = pltpu.async_copy(y_hbm.at[idxv.at[0]], rowsy, sem)

            def gseg(g, _):
                e = jnp.minimum(base + g * LANES + iot, E - 1)
                pos = jnp.zeros((LANES,), jnp.int32)
                for bb in bits:
                    cand = pos + bb
                    rsc = plsc.load_gather(rsv, [jnp.minimum(cand, N)])
                    pos = jnp.where((cand <= N) & (rsc <= e), cand, pos)
                segv[0, pl.ds(g * LANES, LANES)] = pos
                return 0

            lax.fori_loop(0, ECH2 // LANES, gseg, 0)
            cpy.wait()
            pltpu.sync_copy(rowsy, aggy_hbm.at[pl.ds(base, ECH2)])
            pltpu.async_copy(x_hbm.at[segv.at[0]], rowsx, sem).wait()
            pltpu.sync_copy(rowsx, aggx_hbm.at[pl.ds(base, ECH2)])

    return k(y16, x16, nbrp, rsp)


def _tc_mlp(a8y, a8x, w1aB, w1bB, b1B, w2B, b2B, EP):
    grid = EP // TLE
    R8 = TLE // 8

    def body(ya_ref, xa_ref, w1a_ref, w1b_ref, b1_ref, w2_ref, b2_ref, o_ref):
        hp = lax.dot_general(
            ya_ref[...], w1a_ref[...], (((1,), (0,)), ((), ())),
            preferred_element_type=jnp.float32)
        hp = hp + lax.dot_general(
            xa_ref[...], w1b_ref[...], (((1,), (0,)), ((), ())),
            preferred_element_type=jnp.float32)
        h8 = jax.nn.gelu(hp + b1_ref[...])
        h2 = h8.reshape(TLE // 2, 128)
        o_ref[...] = lax.dot_general(
            h2, w2_ref[...], (((1,), (0,)), ((), ())),
            preferred_element_type=jnp.float32) + b2_ref[...]

    return pl.pallas_call(
        body,
        grid=(grid,),
        in_specs=[
            pl.BlockSpec((R8, 128), lambda i: (i, 0)),
            pl.BlockSpec((R8, 128), lambda i: (i, 0)),
            pl.BlockSpec((128, 512), lambda i: (0, 0)),
            pl.BlockSpec((128, 512), lambda i: (0, 0)),
            pl.BlockSpec((1, 512), lambda i: (0, 0)),
            pl.BlockSpec((128, 128), lambda i: (0, 0)),
            pl.BlockSpec((1, 128), lambda i: (0, 0)),
        ],
        out_specs=pl.BlockSpec((TLE // 2, 128), lambda i: (i, 0)),
        out_shape=jax.ShapeDtypeStruct((EP // 2, 128), jnp.float32),
    )(a8y, a8x, w1aB, w1bB, b1B, w2B, b2B)


def _reduce(k1d, fy, nbrp, rsp, N, EP, NPW, NWPAD):
    NCC = NPW // NCH
    RSW = NCH + 48

    @functools.partial(
        pl.kernel,
        out_type=jax.ShapeDtypeStruct((NWPAD * 64,), jnp.float32),
        mesh=_mesh(),
        compiler_params=_SC_PARAMS,
        scratch_types=[
            pltpu.VMEM((RSW,), jnp.int32),
            pltpu.VMEM((ECH * 64,), jnp.float32),
            pltpu.VMEM((1, ECH + 16), jnp.int32),
            pltpu.VMEM((ECH + 16, 64), jnp.float32),
            pltpu.VMEM((NCH * 64,), jnp.float32),
            pltpu.SemaphoreType.DMA,
        ],
    )
    def k(k_hbm, fy_hbm, nbr_hbm, rs_hbm, out_hbm,
          rsv, kbuf, idxv, fybuf, obuf, sem):
        w = _wid()
        n0 = w * NPW
        z16 = jnp.zeros((LANES,), jnp.float32)

        @pl.loop(0, NCC)
        def _(ncc):
            cs = n0 + ncc * NCH
            bs = (cs // 16) * 16
            h0 = cs - bs
            pltpu.sync_copy(rs_hbm.at[pl.ds(bs, RSW)], rsv)
            e0 = _sr(rsv, h0)
            e1c = _sr(rsv, h0 + NCH)
            nchunks = lax.max((e1c - e0 + ECH - 1) // ECH, 1)

            def schunk(s, carry):
                base = e0 + s * ECH
                cend = jnp.minimum(base + ECH, e1c)
                ib = (base // 16) * 16
                pltpu.sync_copy(nbr_hbm.at[pl.ds(ib, ECH + 16)], idxv.at[0])
                pltpu.async_copy(fy_hbm.at[idxv.at[0]], fybuf, sem).wait()
                pltpu.sync_copy(k_hbm.at[pl.ds(base * 64, ECH * 64)], kbuf)

                def edge_acc(elo, ehi, acc):
                    def eb(e, ac):
                        ko = (e - base) * 64
                        fr = e - ib
                        a0, a1, a2, a3 = ac
                        a0 = a0 + kbuf[pl.ds(ko, 16)] * fybuf[fr, pl.ds(0, 16)]
                        a1 = a1 + kbuf[pl.ds(ko + 16, 16)] * fybuf[fr, pl.ds(16, 16)]
                        a2 = a2 + kbuf[pl.ds(ko + 32, 16)] * fybuf[fr, pl.ds(32, 16)]
                        a3 = a3 + kbuf[pl.ds(ko + 48, 16)] * fybuf[fr, pl.ds(48, 16)]
                        return (a0, a1, a2, a3)

                    return lax.fori_loop(elo, ehi, eb, acc)

                def wcond(c):
                    return (c[0] < cs + NCH) & (_sr(rsv, c[0] + 1 - bs) <= cend)

                def wbody(c):
                    nn = c[0]
                    a = _sr(rsv, nn - bs)
                    b = _sr(rsv, nn + 1 - bs)
                    acc = edge_acc(jnp.maximum(a, base), b, c[1:])
                    sc = (jnp.ones((LANES,), jnp.float32)
                          / jnp.full((LANES,),
                                     jnp.maximum(b - a, 1).astype(jnp.float32)))
                    o = (nn - cs) * 64
                    obuf[pl.ds(o, 16)] = acc[0] * sc
                    obuf[pl.ds(o + 16, 16)] = acc[1] * sc
                    obuf[pl.ds(o + 32, 16)] = acc[2] * sc
                    obuf[pl.ds(o + 48, 16)] = acc[3] * sc
                    return (nn + 1, z16, z16, z16, z16)

                c = lax.while_loop(wcond, wbody, (carry[0],) + carry[1:])
                nn = c[0]
                a = _sr(rsv, nn - bs)
                st = jnp.maximum(a, base)
                en = jnp.maximum(cend, st)
                acc = edge_acc(st, en, c[1:])
                return (nn,) + acc

            lax.fori_loop(0, nchunks, schunk, (cs, z16, z16, z16, z16))
            pltpu.sync_copy(obuf, out_hbm.at[pl.ds(cs * 64, NCH * 64)])

    return k(k1d, fy, nbrp, rsp)


def kernel(y, x, f_y, W1, b1, W2, b2, neighbors_index, neighbors_row_splits):
    N = y.shape[0]
    E = neighbors_index.shape[0]

    EP = _rup(E + 1024, NW * ECH2)
    assert EP % TLE == 0

    NPW = _rup(_cdiv(N, NW), NCH)
    NWPAD = NW * NPW

    nbr = neighbors_index.astype(jnp.int32)
    rs = neighbors_row_splits.astype(jnp.int32)

    y16 = jnp.pad(y.astype(jnp.float32), ((0, 0), (0, 13)))
    x16 = jnp.pad(x.astype(jnp.float32), ((0, 0), (0, 13)))
    nbrp = jnp.pad(nbr, (0, EP - E))
    rsp = jnp.pad(rs, (0, NWPAD + 416 - (N + 1)), constant_values=E)

    w1a = jnp.pad(W1[:3].astype(jnp.float32), ((0, 13), (0, 0)))
    w1b = jnp.pad(W1[3:].astype(jnp.float32), ((0, 13), (0, 0)))
    eye8 = jnp.eye(8, dtype=jnp.float32)
    eye2 = jnp.eye(2, dtype=jnp.float32)
    w1aB = jnp.kron(eye8, w1a)
    w1bB = jnp.kron(eye8, w1b)
    b1B = jnp.tile(b1.astype(jnp.float32), 8).reshape(1, 512)
    w2B = jnp.kron(eye2, W2.astype(jnp.float32))
    b2B = jnp.tile(b2.astype(jnp.float32), 2).reshape(1, 128)

    aggy, aggx = _gather_phase(y16, x16, nbrp, rsp, N, E, EP)
    a8y = aggy.reshape(EP // 8, 128)
    a8x = aggx.reshape(EP // 8, 128)
    kk = _tc_mlp(a8y, a8x, w1aB, w1bB, b1B, w2B, b2B, EP)
    out_flat = _reduce(kk.reshape(EP * 64), f_y.astype(jnp.float32),
                       nbrp, rsp, N, EP, NPW, NWPAD)
    return out_flat.reshape(NWPAD, 64)[:N]


# speedup vs baseline: 12.6399x; 1.2959x over previous
import functools

import jax
import jax.numpy as jnp
from jax import lax
from jax.experimental import pallas as pl
from jax.experimental.pallas import tpu as pltpu
from jax.experimental.pallas import tpu_sc as plsc

NW = 32
LANES = 16
NCH = 196
ECH = 384
ECH2 = 512
TLE = 4096


def _cdiv(a, b):
    return (a + b - 1) // b


def _rup(a, b):
    return _cdiv(a, b) * b


def _wid():
    return lax.axis_index("s") * 2 + lax.axis_index("c")


def _sr(ref, i):
    return ref[pl.ds(i, LANES)][0]


def _mesh():
    return plsc.VectorSubcoreMesh(core_axis_name="c", subcore_axis_name="s")


_SC_PARAMS = pltpu.CompilerParams(needs_layout_passes=False,
                                 use_tc_tiling_on_sc=False)


def _gather_phase(y16, x16, nbrp, rsp, N, E, EP):
    EPW = EP // NW
    nchunk = EPW // ECH2
    RSP = _rup(N + 1, LANES)
    bits = []
    b = 1
    while b <= N:
        b *= 2
    while b >= 1:
        bits.append(b)
        b //= 2

    @functools.partial(
        pl.kernel,
        out_type=(jax.ShapeDtypeStruct((EP, 16), jnp.float32),
                  jax.ShapeDtypeStruct((EP, 16), jnp.float32)),
        mesh=_mesh(),
        compiler_params=_SC_PARAMS,
        scratch_types=[
            pltpu.VMEM((RSP,), jnp.int32),
            pltpu.VMEM((2, ECH2), jnp.int32),
            pltpu.VMEM((2, ECH2), jnp.int32),
            pltpu.VMEM((2, ECH2, 16), jnp.float32),
            pltpu.VMEM((2, ECH2, 16), jnp.float32),
            pltpu.SemaphoreType.DMA((2,)),
            pltpu.SemaphoreType.DMA((2,)),
            pltpu.SemaphoreType.DMA((2,)),
            pltpu.SemaphoreType.DMA((2,)),
        ],
    )
    def k(y_hbm, x_hbm, nbr_hbm, rs_hbm, aggy_hbm, aggx_hbm,
          rsv, idxv, segv, rowsy, rowsx, semy, semx, semoy, semox):
        w = _wid()
        base0 = w * EPW
        pltpu.sync_copy(rs_hbm.at[pl.ds(0, RSP)], rsv)
        iot = jnp.arange(LANES, dtype=jnp.int32)

        def idx_stage(t, sl):
            pltpu.sync_copy(nbr_hbm.at[pl.ds(base0 + t * ECH2, ECH2)],
                            idxv.at[sl])

        def y_copy(t, sl):
            return pltpu.make_async_copy(
                y_hbm.at[idxv.at[sl]], rowsy.at[sl], semy.at[sl])

        def x_copy(t, sl):
            return pltpu.make_async_copy(
                x_hbm.at[segv.at[sl]], rowsx.at[sl], semx.at[sl])

        def oy_copy(t, sl):
            return pltpu.make_async_copy(
                rowsy.at[sl], aggy_hbm.at[pl.ds(base0 + t * ECH2, ECH2)],
                semoy.at[sl])

        def ox_copy(t, sl):
            return pltpu.make_async_copy(
                rowsx.at[sl], aggx_hbm.at[pl.ds(base0 + t * ECH2, ECH2)],
                semox.at[sl])

        def seg_compute(t, sl):
            base = base0 + t * ECH2

            def gseg(g, _):
                e = jnp.minimum(base + g * LANES + iot, E - 1)
                pos = jnp.zeros((LANES,), jnp.int32)
                for bb in bits:
                    cand = pos + bb
                    rsc = plsc.load_gather(rsv, [jnp.minimum(cand, N)])
                    pos = jnp.where((cand <= N) & (rsc <= e), cand, pos)
                segv[sl, pl.ds(g * LANES, LANES)] = pos
                return 0

            lax.fori_loop(0, ECH2 // LANES, gseg, 0)

        idx_stage(0, 0)
        y_copy(0, 0).start()

        @pl.loop(0, nchunk)
        def _(t):
            c = t % 2
            d = 1 - c
            seg_compute(t, c)
            y_copy(t, c).wait()

            @pl.when(t >= 2)
            def _():
                ox_copy(t, c).wait()

            x_copy(t, c).start()

            @pl.when(t + 1 < nchunk)
            def _():
                @pl.when(t >= 1)
                def _():
                    oy_copy(t, d).wait()

                idx_stage(t + 1, d)
                y_copy(t + 1, d).start()

            oy_copy(t, c).start()
            x_copy(t, c).wait()
            ox_copy(t, c).start()

        oy_copy(nchunk - 2, nchunk % 2).wait()
        oy_copy(nchunk - 1, (nchunk - 1) % 2).wait()
        ox_copy(nchunk - 2, nchunk % 2).wait()
        ox_copy(nchunk - 1, (nchunk - 1) % 2).wait()

    return k(y16, x16, nbrp, rsp)


def _tc_mlp(a8y, a8x, w1aB, w1bB, b1B, w2B, b2B, EP):
    grid = EP // TLE
    R8 = TLE // 8

    def body(ya_ref, xa_ref, w1a_ref, w1b_ref, b1_ref, w2_ref, b2_ref, o_ref):
        hp = lax.dot_general(
            ya_ref[...], w1a_ref[...], (((1,), (0,)), ((), ())),
            preferred_element_type=jnp.float32)
        hp = hp + lax.dot_general(
            xa_ref[...], w1b_ref[...], (((1,), (0,)), ((), ())),
            preferred_element_type=jnp.float32)
        h8 = jax.nn.gelu(hp + b1_ref[...])
        h2 = h8.reshape(TLE // 2, 128)
        o_ref[...] = lax.dot_general(
            h2, w2_ref[...], (((1,), (0,)), ((), ())),
            preferred_element_type=jnp.float32) + b2_ref[...]

    return pl.pallas_call(
        body,
        grid=(grid,),
        in_specs=[
            pl.BlockSpec((R8, 128), lambda i: (i, 0)),
            pl.BlockSpec((R8, 128), lambda i: (i, 0)),
            pl.BlockSpec((128, 512), lambda i: (0, 0)),
            pl.BlockSpec((128, 512), lambda i: (0, 0)),
            pl.BlockSpec((1, 512), lambda i: (0, 0)),
            pl.BlockSpec((128, 128), lambda i: (0, 0)),
            pl.BlockSpec((1, 128), lambda i: (0, 0)),
        ],
        out_specs=pl.BlockSpec((TLE // 2, 128), lambda i: (i, 0)),
        out_shape=jax.ShapeDtypeStruct((EP // 2, 128), jnp.float32),
    )(a8y, a8x, w1aB, w1bB, b1B, w2B, b2B)


def _reduce(k1d, fy, nbrp, rsp, N, EP, NPW, NWPAD):
    NCC = NPW // NCH
    RSW = NCH + 48

    @functools.partial(
        pl.kernel,
        out_type=jax.ShapeDtypeStruct((NWPAD * 64,), jnp.float32),
        mesh=_mesh(),
        compiler_params=_SC_PARAMS,
        scratch_types=[
            pltpu.VMEM((RSW,), jnp.int32),
            pltpu.VMEM((2, ECH * 64), jnp.float32),
            pltpu.VMEM((2, ECH + 16), jnp.int32),
            pltpu.VMEM((2, ECH + 16, 64), jnp.float32),
            pltpu.VMEM((NCH * 64,), jnp.float32),
            pltpu.SemaphoreType.DMA((2,)),
            pltpu.SemaphoreType.DMA((2,)),
        ],
    )
    def k(k_hbm, fy_hbm, nbr_hbm, rs_hbm, out_hbm,
          rsv, kbuf, idxv, fybuf, obuf, semf, semk):
        w = _wid()
        n0 = w * NPW
        z16 = jnp.zeros((LANES,), jnp.float32)

        @pl.loop(0, NCC)
        def _(ncc):
            cs = n0 + ncc * NCH
            bs = (cs // 16) * 16
            h0 = cs - bs
            pltpu.sync_copy(rs_hbm.at[pl.ds(bs, RSW)], rsv)
            e0 = _sr(rsv, h0)
            e1c = _sr(rsv, h0 + NCH)
            nchunks = lax.max((e1c - e0 + ECH - 1) // ECH, 1)

            def fy_copy(base, sl):
                return pltpu.make_async_copy(
                    fy_hbm.at[idxv.at[sl]], fybuf.at[sl], semf.at[sl])

            def k_copy(base, sl):
                return pltpu.make_async_copy(
                    k_hbm.at[pl.ds(base * 64, ECH * 64)], kbuf.at[sl],
                    semk.at[sl])

            def stage(base, sl):
                ib = (base // 16) * 16
                pltpu.sync_copy(nbr_hbm.at[pl.ds(ib, ECH + 16)], idxv.at[sl])
                fy_copy(base, sl).start()
                k_copy(base, sl).start()

            stage(e0, 0)

            def schunk(s, carry):
                c = s % 2
                base = e0 + s * ECH
                cend = jnp.minimum(base + ECH, e1c)
                ib = (base // 16) * 16

                @pl.when(s + 1 < nchunks)
                def _():
                    stage(base + ECH, 1 - c)

                fy_copy(base, c).wait()
                k_copy(base, c).wait()

                def edge_acc(elo, ehi, acc):
                    def eb(e, ac):
                        ko = (e - base) * 64
                        fr = e - ib
                        a0, a1, a2, a3 = ac
                        a0 = a0 + (kbuf[c, pl.ds(ko, 16)]
                                   * fybuf[c, fr, pl.ds(0, 16)])
                        a1 = a1 + (kbuf[c, pl.ds(ko + 16, 16)]
                                   * fybuf[c, fr, pl.ds(16, 16)])
                        a2 = a2 + (kbuf[c, pl.ds(ko + 32, 16)]
                                   * fybuf[c, fr, pl.ds(32, 16)])
                        a3 = a3 + (kbuf[c, pl.ds(ko + 48, 16)]
                                   * fybuf[c, fr, pl.ds(48, 16)])
                        return (a0, a1, a2, a3)

                    return lax.fori_loop(elo, ehi, eb, acc)

                def wcond(cc):
                    return (cc[0] < cs + NCH) & (_sr(rsv, cc[0] + 1 - bs) <= cend)

                def wbody(cc):
                    nn = cc[0]
                    a = _sr(rsv, nn - bs)
                    b = _sr(rsv, nn + 1 - bs)
                    acc = edge_acc(jnp.maximum(a, base), b, cc[1:])
                    sc = (jnp.ones((LANES,), jnp.float32)
                          / jnp.full((LANES,),
                                     jnp.maximum(b - a, 1).astype(jnp.float32)))
                    o = (nn - cs) * 64
                    obuf[pl.ds(o, 16)] = acc[0] * sc
                    obuf[pl.ds(o + 16, 16)] = acc[1] * sc
                    obuf[pl.ds(o + 32, 16)] = acc[2] * sc
                    obuf[pl.ds(o + 48, 16)] = acc[3] * sc
                    return (nn + 1, z16, z16, z16, z16)

                cc = lax.while_loop(wcond, wbody, (carry[0],) + carry[1:])
                nn = cc[0]
                a = _sr(rsv, nn - bs)
                st = jnp.maximum(a, base)
                en = jnp.maximum(cend, st)
                acc = edge_acc(st, en, cc[1:])
                return (nn,) + acc

            lax.fori_loop(0, nchunks, schunk, (cs, z16, z16, z16, z16))
            pltpu.sync_copy(obuf, out_hbm.at[pl.ds(cs * 64, NCH * 64)])

    return k(k1d, fy, nbrp, rsp)


def kernel(y, x, f_y, W1, b1, W2, b2, neighbors_index, neighbors_row_splits):
    N = y.shape[0]
    E = neighbors_index.shape[0]

    EP = _rup(E + 1024, NW * ECH2)
    assert EP % TLE == 0

    NPW = _rup(_cdiv(N, NW), NCH)
    NWPAD = NW * NPW

    nbr = neighbors_index.astype(jnp.int32)
    rs = neighbors_row_splits.astype(jnp.int32)

    y16 = jnp.pad(y.astype(jnp.float32), ((0, 0), (0, 13)))
    x16 = jnp.pad(x.astype(jnp.float32), ((0, 0), (0, 13)))
    nbrp = jnp.pad(nbr, (0, EP - E))
    rsp = jnp.pad(rs, (0, NWPAD + 512 - (N + 1)), constant_values=E)

    w1a = jnp.pad(W1[:3].astype(jnp.float32), ((0, 13), (0, 0)))
    w1b = jnp.pad(W1[3:].astype(jnp.float32), ((0, 13), (0, 0)))
    eye8 = jnp.eye(8, dtype=jnp.float32)
    eye2 = jnp.eye(2, dtype=jnp.float32)
    w1aB = jnp.kron(eye8, w1a)
    w1bB = jnp.kron(eye8, w1b)
    b1B = jnp.tile(b1.astype(jnp.float32), 8).reshape(1, 512)
    w2B = jnp.kron(eye2, W2.astype(jnp.float32))
    b2B = jnp.tile(b2.astype(jnp.float32), 2).reshape(1, 128)

    aggy, aggx = _gather_phase(y16, x16, nbrp, rsp, N, E, EP)
    a8y = aggy.reshape(EP // 8, 128)
    a8x = aggx.reshape(EP // 8, 128)
    kk = _tc_mlp(a8y, a8x, w1aB, w1bB, b1B, w2B, b2B, EP)
    out_flat = _reduce(kk.reshape(EP * 64), f_y.astype(jnp.float32),
                       nbrp, rsp, N, EP, NPW, NWPAD)
    return out_flat.reshape(NWPAD, 64)[:N]


# speedup vs baseline: 12.6860x; 1.0036x over previous
import functools

import jax
import jax.numpy as jnp
from jax import lax
from jax.experimental import pallas as pl
from jax.experimental.pallas import tpu as pltpu
from jax.experimental.pallas import tpu_sc as plsc

NW = 32
LANES = 16
NCH = 196
ECH = 384
ECH2 = 1024
TLE = 4096


def _cdiv(a, b):
    return (a + b - 1) // b


def _rup(a, b):
    return _cdiv(a, b) * b


def _wid():
    return lax.axis_index("s") * 2 + lax.axis_index("c")


def _sr(ref, i):
    return ref[pl.ds(i, LANES)][0]


def _mesh():
    return plsc.VectorSubcoreMesh(core_axis_name="c", subcore_axis_name="s")


_SC_PARAMS = pltpu.CompilerParams(needs_layout_passes=False,
                                 use_tc_tiling_on_sc=False)


def _gather_phase(y16, x16, nbrp, rsp, N, E, EP):
    EPW = EP // NW
    nchunk = EPW // ECH2
    RSP = _rup(N + 1, LANES)
    bits = []
    b = 1
    while b <= N:
        b *= 2
    while b >= 1:
        bits.append(b)
        b //= 2

    @functools.partial(
        pl.kernel,
        out_type=(jax.ShapeDtypeStruct((EP, 16), jnp.float32),
                  jax.ShapeDtypeStruct((EP, 16), jnp.float32)),
        mesh=_mesh(),
        compiler_params=_SC_PARAMS,
        scratch_types=[
            pltpu.VMEM((RSP,), jnp.int32),
            pltpu.VMEM((2, ECH2), jnp.int32),
            pltpu.VMEM((2, ECH2), jnp.int32),
            pltpu.VMEM((2, ECH2, 16), jnp.float32),
            pltpu.VMEM((2, ECH2, 16), jnp.float32),
            pltpu.SemaphoreType.DMA((2,)),
            pltpu.SemaphoreType.DMA((2,)),
            pltpu.SemaphoreType.DMA((2,)),
            pltpu.SemaphoreType.DMA((2,)),
        ],
    )
    def k(y_hbm, x_hbm, nbr_hbm, rs_hbm, aggy_hbm, aggx_hbm,
          rsv, idxv, segv, rowsy, rowsx, semy, semx, semoy, semox):
        w = _wid()
        base0 = w * EPW
        pltpu.sync_copy(rs_hbm.at[pl.ds(0, RSP)], rsv)
        iot = jnp.arange(LANES, dtype=jnp.int32)

        def idx_stage(t, sl):
            pltpu.sync_copy(nbr_hbm.at[pl.ds(base0 + t * ECH2, ECH2)],
                            idxv.at[sl])

        def y_copy(t, sl):
            return pltpu.make_async_copy(
                y_hbm.at[idxv.at[sl]], rowsy.at[sl], semy.at[sl])

        def x_copy(t, sl):
            return pltpu.make_async_copy(
                x_hbm.at[segv.at[sl]], rowsx.at[sl], semx.at[sl])

        def oy_copy(t, sl):
            return pltpu.make_async_copy(
                rowsy.at[sl], aggy_hbm.at[pl.ds(base0 + t * ECH2, ECH2)],
                semoy.at[sl])

        def ox_copy(t, sl):
            return pltpu.make_async_copy(
                rowsx.at[sl], aggx_hbm.at[pl.ds(base0 + t * ECH2, ECH2)],
                semox.at[sl])

        def seg_compute(t, sl):
            base = base0 + t * ECH2

            def gseg(g, _):
                e0_ = jnp.minimum(base + 2 * g * LANES + iot, E - 1)
                e1_ = jnp.minimum(base + (2 * g + 1) * LANES + iot, E - 1)
                p0 = jnp.zeros((LANES,), jnp.int32)
                p1 = jnp.zeros((LANES,), jnp.int32)
                for bb in bits:
                    c0 = p0 + bb
                    c1 = p1 + bb
                    r0 = plsc.load_gather(rsv, [jnp.minimum(c0, N)])
                    r1 = plsc.load_gather(rsv, [jnp.minimum(c1, N)])
                    p0 = jnp.where((c0 <= N) & (r0 <= e0_), c0, p0)
                    p1 = jnp.where((c1 <= N) & (r1 <= e1_), c1, p1)
                segv[sl, pl.ds(2 * g * LANES, LANES)] = p0
                segv[sl, pl.ds((2 * g + 1) * LANES, LANES)] = p1
                return 0

            lax.fori_loop(0, ECH2 // LANES // 2, gseg, 0)

        idx_stage(0, 0)
        y_copy(0, 0).start()

        @pl.loop(0, nchunk)
        def _(t):
            c = t % 2
            d = 1 - c
            seg_compute(t, c)
            y_copy(t, c).wait()

            @pl.when(t >= 2)
            def _():
                ox_copy(t, c).wait()

            x_copy(t, c).start()

            @pl.when(t + 1 < nchunk)
            def _():
                @pl.when(t >= 1)
                def _():
                    oy_copy(t, d).wait()

                idx_stage(t + 1, d)
                y_copy(t + 1, d).start()

            oy_copy(t, c).start()
            x_copy(t, c).wait()
            ox_copy(t, c).start()

        oy_copy(nchunk - 2, nchunk % 2).wait()
        oy_copy(nchunk - 1, (nchunk - 1) % 2).wait()
        ox_copy(nchunk - 2, nchunk % 2).wait()
        ox_copy(nchunk - 1, (nchunk - 1) % 2).wait()

    return k(y16, x16, nbrp, rsp)


def _tc_mlp(a8y, a8x, w1aB, w1bB, b1B, w2B, b2B, EP):
    grid = EP // TLE
    R8 = TLE // 8

    def body(ya_ref, xa_ref, w1a_ref, w1b_ref, b1_ref, w2_ref, b2_ref, o_ref):
        hp = lax.dot_general(
            ya_ref[...], w1a_ref[...], (((1,), (0,)), ((), ())),
            preferred_element_type=jnp.float32)
        hp = hp + lax.dot_general(
            xa_ref[...], w1b_ref[...], (((1,), (0,)), ((), ())),
            preferred_element_type=jnp.float32)
        h8 = jax.nn.gelu(hp + b1_ref[...])
        h2 = h8.reshape(TLE // 2, 128)
        o_ref[...] = lax.dot_general(
            h2, w2_ref[...], (((1,), (0,)), ((), ())),
            preferred_element_type=jnp.float32) + b2_ref[...]

    return pl.pallas_call(
        body,
        grid=(grid,),
        in_specs=[
            pl.BlockSpec((R8, 128), lambda i: (i, 0)),
            pl.BlockSpec((R8, 128), lambda i: (i, 0)),
            pl.BlockSpec((128, 512), lambda i: (0, 0)),
            pl.BlockSpec((128, 512), lambda i: (0, 0)),
            pl.BlockSpec((1, 512), lambda i: (0, 0)),
            pl.BlockSpec((128, 128), lambda i: (0, 0)),
            pl.BlockSpec((1, 128), lambda i: (0, 0)),
        ],
        out_specs=pl.BlockSpec((TLE // 2, 128), lambda i: (i, 0)),
        out_shape=jax.ShapeDtypeStruct((EP // 2, 128), jnp.float32),
    )(a8y, a8x, w1aB, w1bB, b1B, w2B, b2B)


def _reduce(k1d, fy, nbrp, rsp, N, EP, NPW, NWPAD):
    NCC = NPW // NCH
    RSW = NCH + 48

    @functools.partial(
        pl.kernel,
        out_type=jax.ShapeDtypeStruct((NWPAD * 64,), jnp.float32),
        mesh=_mesh(),
        compiler_params=_SC_PARAMS,
        scratch_types=[
            pltpu.VMEM((RSW,), jnp.int32),
            pltpu.VMEM((2, ECH * 64), jnp.float32),
            pltpu.VMEM((2, ECH + 16), jnp.int32),
            pltpu.VMEM((2, ECH + 16, 64), jnp.float32),
            pltpu.VMEM((NCH * 64,), jnp.float32),
            pltpu.SemaphoreType.DMA((2,)),
            pltpu.SemaphoreType.DMA((2,)),
        ],
    )
    def k(k_hbm, fy_hbm, nbr_hbm, rs_hbm, out_hbm,
          rsv, kbuf, idxv, fybuf, obuf, semf, semk):
        w = _wid()
        n0 = w * NPW
        z16 = jnp.zeros((LANES,), jnp.float32)

        @pl.loop(0, NCC)
        def _(ncc):
            cs = n0 + ncc * NCH
            bs = (cs // 16) * 16
            h0 = cs - bs
            pltpu.sync_copy(rs_hbm.at[pl.ds(bs, RSW)], rsv)
            e0 = _sr(rsv, h0)
            e1c = _sr(rsv, h0 + NCH)
            nchunks = lax.max((e1c - e0 + ECH - 1) // ECH, 1)

            def fy_copy(base, sl):
                return pltpu.make_async_copy(
                    fy_hbm.at[idxv.at[sl]], fybuf.at[sl], semf.at[sl])

            def k_copy(base, sl):
                return pltpu.make_async_copy(
                    k_hbm.at[pl.ds(base * 64, ECH * 64)], kbuf.at[sl],
                    semk.at[sl])

            def stage(base, sl):
                ib = (base // 16) * 16
                pltpu.sync_copy(nbr_hbm.at[pl.ds(ib, ECH + 16)], idxv.at[sl])
                fy_copy(base, sl).start()
                k_copy(base, sl).start()

            stage(e0, 0)

            def schunk(s, carry):
                c = s % 2
                base = e0 + s * ECH
                cend = jnp.minimum(base + ECH, e1c)
                ib = (base // 16) * 16

                @pl.when(s + 1 < nchunks)
                def _():
                    stage(base + ECH, 1 - c)

                fy_copy(base, c).wait()
                k_copy(base, c).wait()

                def edge_acc(elo, ehi, acc):
                    def eb(e, ac):
                        ko = (e - base) * 64
                        fr = e - ib
                        a0, a1, a2, a3 = ac
                        a0 = a0 + (kbuf[c, pl.ds(ko, 16)]
                                   * fybuf[c, fr, pl.ds(0, 16)])
                        a1 = a1 + (kbuf[c, pl.ds(ko + 16, 16)]
                                   * fybuf[c, fr, pl.ds(16, 16)])
                        a2 = a2 + (kbuf[c, pl.ds(ko + 32, 16)]
                                   * fybuf[c, fr, pl.ds(32, 16)])
                        a3 = a3 + (kbuf[c, pl.ds(ko + 48, 16)]
                                   * fybuf[c, fr, pl.ds(48, 16)])
                        return (a0, a1, a2, a3)

                    return lax.fori_loop(elo, ehi, eb, acc)

                def wcond(cc):
                    return (cc[0] < cs + NCH) & (_sr(rsv, cc[0] + 1 - bs) <= cend)

                def wbody(cc):
                    nn = cc[0]
                    a = _sr(rsv, nn - bs)
                    b = _sr(rsv, nn + 1 - bs)
                    acc = edge_acc(jnp.maximum(a, base), b, cc[1:])
                    sc = (jnp.ones((LANES,), jnp.float32)
                          / jnp.full((LANES,),
                                     jnp.maximum(b - a, 1).astype(jnp.float32)))
                    o = (nn - cs) * 64
                    obuf[pl.ds(o, 16)] = acc[0] * sc
                    obuf[pl.ds(o + 16, 16)] = acc[1] * sc
                    obuf[pl.ds(o + 32, 16)] = acc[2] * sc
                    obuf[pl.ds(o + 48, 16)] = acc[3] * sc
                    return (nn + 1, z16, z16, z16, z16)

                cc = lax.while_loop(wcond, wbody, (carry[0],) + carry[1:])
                nn = cc[0]
                a = _sr(rsv, nn - bs)
                st = jnp.maximum(a, base)
                en = jnp.maximum(cend, st)
                acc = edge_acc(st, en, cc[1:])
                return (nn,) + acc

            lax.fori_loop(0, nchunks, schunk, (cs, z16, z16, z16, z16))
            pltpu.sync_copy(obuf, out_hbm.at[pl.ds(cs * 64, NCH * 64)])

    return k(k1d, fy, nbrp, rsp)


def kernel(y, x, f_y, W1, b1, W2, b2, neighbors_index, neighbors_row_splits):
    N = y.shape[0]
    E = neighbors_index.shape[0]

    EP = _rup(E + 1024, NW * ECH2)
    assert EP % TLE == 0

    NPW = _rup(_cdiv(N, NW), NCH)
    NWPAD = NW * NPW

    nbr = neighbors_index.astype(jnp.int32)
    rs = neighbors_row_splits.astype(jnp.int32)

    y16 = jnp.pad(y.astype(jnp.float32), ((0, 0), (0, 13)))
    x16 = jnp.pad(x.astype(jnp.float32), ((0, 0), (0, 13)))
    nbrp = jnp.pad(nbr, (0, EP - E))
    rsp = jnp.pad(rs, (0, NWPAD + 512 - (N + 1)), constant_values=E)

    w1a = jnp.pad(W1[:3].astype(jnp.float32), ((0, 13), (0, 0)))
    w1b = jnp.pad(W1[3:].astype(jnp.float32), ((0, 13), (0, 0)))
    eye8 = jnp.eye(8, dtype=jnp.float32)
    eye2 = jnp.eye(2, dtype=jnp.float32)
    w1aB = jnp.kron(eye8, w1a)
    w1bB = jnp.kron(eye8, w1b)
    b1B = jnp.tile(b1.astype(jnp.float32), 8).reshape(1, 512)
    w2B = jnp.kron(eye2, W2.astype(jnp.float32))
    b2B = jnp.tile(b2.astype(jnp.float32), 2).reshape(1, 128)

    aggy, aggx = _gather_phase(y16, x16, nbrp, rsp, N, E, EP)
    a8y = aggy.reshape(EP // 8, 128)
    a8x = aggx.reshape(EP // 8, 128)
    kk = _tc_mlp(a8y, a8x, w1aB, w1bB, b1B, w2B, b2B, EP)
    out_flat = _reduce(kk.reshape(EP * 64), f_y.astype(jnp.float32),
                       nbrp, rsp, N, EP, NPW, NWPAD)
    return out_flat.reshape(NWPAD, 64)[:N]
